# Initial kernel scaffold; baseline (speedup 1.0000x reference)
import jax
import jax.numpy as jnp
from jax.experimental import pallas as pl


def kernel(input_ids):
    raise NotImplementedError("write your pallas kernel here")


# speedup vs baseline: 1.1736x; 1.1736x over previous
import functools

import jax
import jax.numpy as jnp
from jax import lax
from jax.experimental import pallas as pl
from jax.experimental.pallas import tpu as pltpu
from jax.experimental.pallas import tpu_sc as plsc

_V = 30522
_SPECIALS = (0, 101, 102, 103)
_L = 16


@functools.cache
def _make_bow_kernel(B: int, T: int):
    NC, NS = 2, 16
    NW = NC * NS
    assert B % (NW * 4) == 0, B
    rows_per_w = B // NW
    rows_per_chunk = 4
    n_chunks = rows_per_w // rows_per_chunk
    t_steps = -(-T // _L)
    chunk_words = rows_per_chunk * _V
    buf_words = -(-chunk_words // _L) * _L
    ids_words = -(-rows_per_chunk * T // _L) * _L + _L

    mesh = plsc.VectorSubcoreMesh(
        core_axis_name="c", subcore_axis_name="s", num_cores=NC, num_subcores=NS
    )

    @functools.partial(
        pl.kernel,
        out_type=jax.ShapeDtypeStruct((B * _V,), jnp.float32),
        mesh=mesh,
        scratch_types=[
            pltpu.VMEM((ids_words,), jnp.int32),
            pltpu.VMEM((buf_words,), jnp.float32),
        ],
        compiler_params=pltpu.CompilerParams(needs_layout_passes=False),
    )
    def bow_kernel(ids_hbm, out_hbm, ids_v, row_v):
        wid = lax.axis_index("s") * NC + lax.axis_index("c")
        zeros16 = jnp.zeros((_L,), jnp.float32)
        ones16 = jnp.ones((_L,), jnp.float32)
        lane = lax.iota(jnp.int32, _L)

        def zbody(i, carry):
            row_v[pl.ds(i * _L, _L)] = zeros16
            return carry

        lax.fori_loop(0, buf_words // _L, zbody, 0)

        row0 = wid * rows_per_w

        def scatter_all(val16):
            for r in range(rows_per_chunk):
                for t in range(t_steps):
                    ids16 = ids_v[pl.ds(r * T + t * _L, _L)]
                    m = ids16 != _SPECIALS[0]
                    for s in _SPECIALS[1:]:
                        m = m & (ids16 != s)
                    if (t + 1) * _L > T:
                        m = m & (lane < (T - t * _L))
                    plsc.store_scatter(row_v, [ids16 + r * _V], val16, mask=m)

        def chunk_body(c, carry):
            crow = row0 + c * rows_per_chunk
            pltpu.sync_copy(
                ids_hbm.at[pl.ds(crow * T, rows_per_chunk * T)],
                ids_v.at[pl.ds(0, rows_per_chunk * T)],
            )
            scatter_all(ones16)
            out_off = pl.multiple_of(crow * _V, 8)
            pltpu.sync_copy(
                row_v.at[pl.ds(0, chunk_words)],
                out_hbm.at[pl.ds(out_off, chunk_words)],
            )
            scatter_all(zeros16)
            return carry

        lax.fori_loop(0, n_chunks, chunk_body, 0)

    return bow_kernel


def kernel(input_ids):
    B, T = input_ids.shape
    ids_flat = input_ids.astype(jnp.int32).reshape(B * T)
    out_flat = _make_bow_kernel(B, T)(ids_flat)
    return out_flat.reshape(B, _V)


# speedup vs baseline: 5.9740x; 5.0902x over previous
import functools

import jax
import jax.numpy as jnp
from jax import lax
from jax.experimental import pallas as pl
from jax.experimental.pallas import tpu as pltpu
from jax.experimental.pallas import tpu_sc as plsc

_V = 30522
_SPECIALS = (0, 101, 102, 103)
_L = 16


@functools.cache
def _make_bow_kernel(B: int, Tp: int):
    NC, NS = 2, 16
    NW = NC * NS
    assert B % (NW * 8) == 0, B
    assert Tp % 128 == 0, Tp
    rows_per_w = B // NW
    groups_per_w = rows_per_w // 8
    id_tiles = Tp // 128
    s_steps = Tp // _L
    n_full_tiles = _V // 128
    half_tiles = n_full_tiles // 2
    tail_lo = n_full_tiles * 128
    tail_n = _V - tail_lo

    mesh = plsc.VectorSubcoreMesh(
        core_axis_name="c", subcore_axis_name="s", num_cores=NC, num_subcores=NS
    )

    @functools.partial(
        pl.kernel,
        out_type=jax.ShapeDtypeStruct((B, _V), jnp.float32),
        mesh=mesh,
        scratch_types=[
            pltpu.VMEM((id_tiles, 8, 128), jnp.int32),
            pltpu.VMEM((half_tiles, 8, 128), jnp.float32),
            pltpu.VMEM((8, 64), jnp.float32),
            pltpu.SemaphoreType.DMA,
        ],
        compiler_params=pltpu.CompilerParams(needs_layout_passes=False),
    )
    def bow_kernel(ids_hbm, out_hbm, ids_v, blk_v, tail_v, sem):
        wid = lax.axis_index("s") * NC + lax.axis_index("c")
        zeros16 = jnp.zeros((_L,), jnp.float32)
        ones16 = jnp.ones((_L,), jnp.float32)

        def zblk(tl, carry):
            for r in range(8):
                for i in range(128 // _L):
                    blk_v[tl, r, pl.ds(i * _L, _L)] = zeros16
            return carry

        lax.fori_loop(0, half_tiles, zblk, 0)
        for r in range(8):
            for i in range(64 // _L):
                tail_v[r, pl.ds(i * _L, _L)] = zeros16

        row0 = wid * rows_per_w

        def token_vec(r, s):
            return ids_v[s // (128 // _L), r, pl.ds((s % (128 // _L)) * _L, _L)]

        def special_mask(ids16, m):
            for sp in _SPECIALS:
                m = m & (ids16 != sp)
            return m

        def scatter_window(t0, val16):
            lo = t0 * 128
            hi = lo + half_tiles * 128

            def body(r, carry):
                r16 = jnp.full((_L,), 0, jnp.int32) + r
                for s in range(s_steps):
                    ids16 = token_vec(r, s)
                    m = special_mask(ids16, (ids16 >= lo) & (ids16 < hi))
                    tl16 = lax.shift_right_logical(ids16, 7) - t0
                    sub16 = lax.bitwise_and(ids16, 127)
                    plsc.store_scatter(blk_v, [tl16, r16, sub16], val16, mask=m)
                return carry

            lax.fori_loop(0, 8, body, 0)

        def scatter_tail(val16):
            def body(r, carry):
                r16 = jnp.full((_L,), 0, jnp.int32) + r
                for s in range(s_steps):
                    ids16 = token_vec(r, s)
                    m = ids16 >= tail_lo
                    plsc.store_scatter(
                        tail_v, [r16, ids16 - tail_lo], val16, mask=m
                    )
                return carry

            lax.fori_loop(0, 8, body, 0)

        def group_body(g, carry):
            grow = row0 + g * 8
            for it in range(id_tiles):
                pltpu.sync_copy(
                    ids_hbm.at[pl.ds(grow, 8), pl.ds(it * 128, 128)],
                    ids_v.at[it],
                )
            for t0 in (0, half_tiles):
                scatter_window(t0, ones16)

                def fire(tl, carry, t0=t0):
                    pltpu.async_copy(
                        blk_v.at[tl],
                        out_hbm.at[pl.ds(grow, 8), pl.ds((t0 + tl) * 128, 128)],
                        sem,
                    )
                    return carry

                lax.fori_loop(0, half_tiles, fire, 0)

                def drain(tl, carry, t0=t0):
                    pltpu.make_async_copy(
                        blk_v.at[tl],
                        out_hbm.at[pl.ds(grow, 8), pl.ds((t0 + tl) * 128, 128)],
                        sem,
                    ).wait()
                    return carry

                lax.fori_loop(0, half_tiles, drain, 0)
                scatter_window(t0, zeros16)
            scatter_tail(ones16)
            for r in range(8):
                pltpu.sync_copy(
                    tail_v.at[r, pl.ds(0, tail_n)],
                    out_hbm.at[grow + r, pl.ds(tail_lo, tail_n)],
                )
            scatter_tail(zeros16)
            return carry

        lax.fori_loop(0, groups_per_w, group_body, 0)

    return bow_kernel


def kernel(input_ids):
    B, T = input_ids.shape
    Tp = -(-T // 128) * 128
    ids_pad = jnp.pad(input_ids.astype(jnp.int32), ((0, 0), (0, Tp - T)))
    return _make_bow_kernel(B, Tp)(ids_pad)


# speedup vs baseline: 14.6276x; 2.4485x over previous
import functools

import jax
import jax.numpy as jnp
from jax import lax
from jax.experimental import pallas as pl
from jax.experimental.pallas import tpu as pltpu
from jax.experimental.pallas import tpu_sc as plsc

_V = 30522
_SPECIALS = (0, 101, 102, 103)
_L = 16
_SENT = 32256
_RB = 1


@functools.cache
def _make_bow_kernel(B: int, T: int, Tp: int):
    NC, NS = 2, 16
    NW = NC * NS
    assert B % (NW * 128) == 0, B
    assert Tp % 128 == 0, Tp
    rows_per_w = B // NW
    n_bgrp = B // 128
    n_vgrp = -(-_V // 8)
    W_FULL = n_vgrp // 64
    LAST_NT = n_vgrp - W_FULL * 64
    id_tiles = Tp // 128
    cap = rows_per_w * (-(-T // _L) * _L) + 64 * _L
    mesh = plsc.VectorSubcoreMesh(
        core_axis_name="c", subcore_axis_name="s", num_cores=NC, num_subcores=NS
    )

    @functools.partial(
        pl.kernel,
        out_type=jax.ShapeDtypeStruct((n_vgrp, n_bgrp, 8, 128), jnp.float32),
        mesh=mesh,
        scratch_types=[
            pltpu.VMEM((id_tiles, 8, 128), jnp.int32),
            pltpu.VMEM((cap,), jnp.int32),
            pltpu.VMEM((64, 8, 128), jnp.float32),
            pltpu.VMEM((80,), jnp.int32),
            pltpu.VMEM((80,), jnp.int32),
            pltpu.VMEM((80,), jnp.int32),
        ],
        compiler_params=pltpu.CompilerParams(needs_layout_passes=False),
    )
    def bow_kernel(ids_hbm, out_hbm, ids_v, lists_v, stg_v, cnt_v, base_v, cur_v):
        wid = lax.axis_index("s") * NC + lax.axis_index("c")
        zeros16 = jnp.zeros((_L,), jnp.float32)
        ones16 = jnp.ones((_L,), jnp.float32)
        izeros16 = jnp.zeros((_L,), jnp.int32)
        lane = lax.iota(jnp.int32, _L)
        row0 = wid * rows_per_w

        def zstg(tl, carry):
            for r in range(8):
                for i in range(128 // _L):
                    stg_v[tl, r, pl.ds(i * _L, _L)] = zeros16
            return carry

        lax.fori_loop(0, 64, zstg, 0)
        for i in range(64 // _L):
            cnt_v[pl.ds(i * _L, _L)] = izeros16

        n_tok_vecs = []
        for t in range(-(-T // _L)):
            tile, sub = (t * _L) // 128, (t * _L) % 128
            valid = min(_L, T - t * _L)
            n_tok_vecs.append((tile, sub, valid))

        def load_clean(g):
            for it in range(id_tiles):
                pltpu.sync_copy(
                    ids_hbm.at[pl.ds(row0 + g * 8, 8), pl.ds(it * 128, 128)],
                    ids_v.at[it],
                )
            for r in range(8):
                for (tile, sub, valid) in n_tok_vecs:
                    ids16 = ids_v[tile, r, pl.ds(sub, _L)]
                    sp = ids16 == _SPECIALS[0]
                    for s in _SPECIALS[1:]:
                        sp = sp | (ids16 == s)
                    ids_v[tile, r, pl.ds(sub, _L)] = jnp.where(
                        sp, jnp.int32(_SENT), ids16
                    )

        def hist_group(g, carry):
            load_clean(g)
            for r in range(8):
                for (tile, sub, valid) in n_tok_vecs:
                    ids16 = ids_v[tile, r, pl.ds(sub, _L)]
                    m = lane < valid if valid < _L else None
                    w16 = lax.shift_right_logical(ids16, 9)
                    cnt16, last16 = plsc.scan_count(w16, m)
                    add16 = cnt16 + (1 - _RB)
                    plsc.addupdate_scatter(cnt_v, [w16], add16, mask=last16)
            return carry

        lax.fori_loop(0, rows_per_w // 8, hist_group, 0)

        carry_s = jnp.int32(0)
        for i in range(64 // _L):
            c16 = cnt_v[pl.ds(i * _L, _L)]
            sz16 = lax.bitwise_and(c16 + (_L - 1), ~(_L - 1))
            inc = plsc.cumsum(sz16)
            b16 = inc - sz16 + carry_s
            base_v[pl.ds(i * _L, _L)] = b16
            cur_v[pl.ds(i * _L, _L)] = b16
            carry_s = carry_s + lax.reduce_sum(sz16, axes=(0,))

        def place_group(g, carry):
            load_clean(g)
            for r in range(8):
                bi = jnp.full((_L,), 0, jnp.int32) + (g * 8 + r)
                for (tile, sub, valid) in n_tok_vecs:
                    ids16 = ids_v[tile, r, pl.ds(sub, _L)]
                    m = lane < valid if valid < _L else None
                    w16 = lax.shift_right_logical(ids16, 9)
                    cnt16, last16 = plsc.scan_count(w16, m)
                    b16 = plsc.load_gather(cur_v, [w16])
                    vg16 = lax.shift_right_logical(ids16, 3)
                    vgl16 = lax.bitwise_and(vg16, 63)
                    vi16 = lax.bitwise_and(ids16, 7)
                    a16 = vgl16 * 1024 + vi16 * 128 + bi
                    slot16 = b16 + cnt16 - _RB
                    plsc.store_scatter(lists_v, [slot16], a16, mask=m)
                    plsc.store_scatter(
                        cur_v, [w16], b16 + cnt16 + (1 - _RB), mask=last16
                    )
            return carry

        lax.fori_loop(0, rows_per_w // 8, place_group, 0)

        def scatter_list(lo, n, val16):
            def body(i, carry):
                a16 = lists_v[pl.ds(lo + i * _L, _L)]
                m = lane < (n - i * _L)
                t16 = lax.shift_right_logical(a16, 10)
                vi16 = lax.bitwise_and(lax.shift_right_logical(a16, 7), 7)
                bi16 = lax.bitwise_and(a16, 127)
                plsc.store_scatter(stg_v, [t16, vi16, bi16], val16, mask=m)
                return carry

            lax.fori_loop(0, (n + _L - 1) // _L, body, 0)

        def drain(win, nt, carry):
            lo = base_v[pl.ds(win, _L)][0]
            n = cnt_v[pl.ds(win, _L)][0]
            scatter_list(lo, n, ones16)
            pltpu.sync_copy(
                stg_v.at[pl.ds(0, nt)],
                out_hbm.at[pl.ds(win * 64, nt), wid],
            )
            scatter_list(lo, n, zeros16)
            return carry

        lax.fori_loop(0, W_FULL, lambda w, c: drain(w, 64, c), 0)
        drain(W_FULL, LAST_NT, 0)

    return bow_kernel


def kernel(input_ids):
    B, T = input_ids.shape
    Tp = -(-T // 128) * 128
    ids_pad = jnp.pad(input_ids.astype(jnp.int32), ((0, 0), (0, Tp - T)))
    out4d = _make_bow_kernel(B, T, Tp)(ids_pad)
    n_vgrp = -(-_V // 8)
    return out4d.transpose(1, 3, 0, 2).reshape(B, n_vgrp * 8)[:, :_V]
